# baseline (speedup 1.0000x reference)
import functools
import math

import jax
import jax.numpy as jnp
from jax import lax
from jax.experimental import pallas as pl
from jax.experimental.pallas import tpu as pltpu
from jax.experimental.pallas import tpu_sc as plsc

N = 10000
E = 320000
D = 128
G = 64
EPS = 1e-5

NC = 2
NS = 16
NW = NC * NS

CHUNK = 128
NBUF = 4
NCHUNK = math.ceil(E / NW / CHUNK / NBUF) * NBUF
PER_TILE = NCHUNK * CHUNK
E_PAD = PER_TILE * NW

NPAD = 10240
ROWS_PER_TILE = NPAD // NS
PAD_DST = NPAD - 1

BN = 2048
GRID = NPAD // BN


def _make_mesh():
    return plsc.VectorSubcoreMesh(core_axis_name="c", subcore_axis_name="s",
                                  num_cores=NC, num_subcores=NS)


@functools.cache
def _make_deg_kernel():
    @functools.partial(
        pl.kernel,
        out_type=jax.ShapeDtypeStruct((NC, NPAD), jnp.float32),
        mesh=_make_mesh(),
        scratch_types=[
            pltpu.VMEM((NCHUNK, CHUNK), jnp.int32),
            pltpu.VMEM((PER_TILE,), jnp.float32),
            pltpu.VMEM((ROWS_PER_TILE,), jnp.float32),
            pltpu.VMEM_SHARED((NPAD,), jnp.float32),
        ],
    )
    def _deg_kernel(c_hbm, w_hbm, out_hbm, c_v, w_v, zbuf, acc):
        cid = lax.axis_index("c")
        sid = lax.axis_index("s")
        wid = cid * NS + sid

        pltpu.sync_copy(c_hbm.at[wid], c_v)
        pltpu.sync_copy(w_hbm.at[wid], w_v)

        def _z(i, _):
            zbuf[pl.ds(i * 16, 16)] = jnp.zeros((16,), jnp.float32)
            return 0
        lax.fori_loop(0, ROWS_PER_TILE // 16, _z, 0)
        pltpu.sync_copy(zbuf,
                        acc.at[pl.ds(sid * ROWS_PER_TILE, ROWS_PER_TILE)])
        plsc.subcore_barrier()

        def _body(j, _):
            pltpu.sync_copy(w_v.at[pl.ds(j * CHUNK, CHUNK)],
                            acc.at[c_v.at[j]], add=True)
            return 0
        lax.fori_loop(0, NCHUNK, _body, 0)
        plsc.subcore_barrier()

        sl = pl.ds(sid * ROWS_PER_TILE, ROWS_PER_TILE)
        pltpu.sync_copy(acc.at[sl], out_hbm.at[cid, sl])

    return _deg_kernel


@functools.cache
def _make_edge_scatter(F):
    @functools.partial(
        pl.kernel,
        out_type=jax.ShapeDtypeStruct((NC, NPAD, F), jnp.float32),
        mesh=_make_mesh(),
        compiler_params=pltpu.CompilerParams(needs_layout_passes=False,
                                             use_tc_tiling_on_sc=False),
        scratch_types=[
            pltpu.VMEM((PER_TILE,), jnp.int32),
            pltpu.VMEM((NCHUNK, CHUNK), jnp.int32),
            pltpu.VMEM((PER_TILE,), jnp.float32),
            pltpu.VMEM((NBUF, CHUNK, F), jnp.float32),
            pltpu.VMEM_SHARED((NPAD, F), jnp.float32),
        ] + [pltpu.SemaphoreType.DMA] * NBUF,
    )
    def _edge_kernel(r_hbm, c_hbm, w_hbm, y_hbm, out_hbm,
                     r_v, c_v, w_v, bufs, acc, *sems):
        cid = lax.axis_index("c")
        sid = lax.axis_index("s")
        wid = cid * NS + sid

        pltpu.sync_copy(r_hbm.at[wid], r_v)
        pltpu.sync_copy(c_hbm.at[wid], c_v)
        pltpu.sync_copy(w_hbm.at[wid], w_v)

        buf0 = bufs.at[0]

        def _z(i, _):
            for k in range(F // 16):
                buf0[i, pl.ds(k * 16, 16)] = jnp.zeros((16,), jnp.float32)
            return 0
        lax.fori_loop(0, CHUNK, _z, 0)
        for b in range(ROWS_PER_TILE // CHUNK):
            sl = pl.ds(sid * ROWS_PER_TILE + b * CHUNK, CHUNK)
            pltpu.sync_copy(buf0, acc.at[sl])
        plsc.subcore_barrier()

        def _gather(j, b):
            r_chunk = r_v.at[pl.ds(j * CHUNK, CHUNK)]
            return pltpu.make_async_copy(y_hbm.at[r_chunk], bufs.at[b],
                                         sems[b])

        for b in range(NBUF):
            _gather(b, b).start()

        def _group(g, _):
            for b in range(NBUF):
                j = g * NBUF + b
                buf = bufs.at[b]
                _gather(j, b).wait()

                def _scale(e, _):
                    idx = jnp.full((16,), j * CHUNK + e, jnp.int32)
                    wv = plsc.load_gather(w_v, [idx])
                    for k in range(F // 16):
                        sl = pl.ds(k * 16, 16)
                        buf[e, sl] = buf[e, sl] * wv
                    return 0
                lax.fori_loop(0, CHUNK, _scale, 0)

                pltpu.sync_copy(buf, acc.at[c_v.at[j]], add=True)

                @pl.when(j + NBUF < NCHUNK)
                def _():
                    _gather(j + NBUF, b).start()
            return 0
        lax.fori_loop(0, NCHUNK // NBUF, _group, 0)
        plsc.subcore_barrier()

        sl = pl.ds(sid * ROWS_PER_TILE, ROWS_PER_TILE)
        pltpu.sync_copy(acc.at[sl], out_hbm.at[cid, sl])

    return _edge_kernel


def _tc1_body(degp_ref, x_ref, w_ref, dinv_ref, y_ref):
    ones = jnp.ones((NC, 1), jnp.float32)
    deg = lax.dot_general(degp_ref[...], ones, (((0,), (0,)), ((), ())),
                          preferred_element_type=jnp.float32, precision=lax.Precision.HIGHEST) + 1.0
    dinv = lax.rsqrt(deg)
    xw = jnp.dot(x_ref[...], w_ref[...], preferred_element_type=jnp.float32, precision=lax.Precision.HIGHEST)
    dinv_ref[...] = dinv
    y_ref[...] = xw * dinv


def _tc_mid_body(p_ref, y_ref, dinv_ref, gs_ref, gb_ref, w_ref, out_ref):
    p = p_ref[0] + p_ref[1] + y_ref[...]
    dinv = dinv_ref[...]
    h = jnp.maximum(p * dinv * gs_ref[...] + gb_ref[...], 0.0)
    out_ref[...] = jnp.dot(h, w_ref[...],
                           preferred_element_type=jnp.float32, precision=lax.Precision.HIGHEST) * dinv


def _tc_final_body(p_ref, y_ref, dinv_ref, gs_ref, gb_ref, batch_ref,
                   fc1w_ref, fc1b_ref, outw_ref, outb_ref, out_ref,
                   sums_s, cnt_s):
    i = pl.program_id(0)

    @pl.when(i == 0)
    def _():
        sums_s[...] = jnp.zeros_like(sums_s)
        cnt_s[...] = jnp.zeros_like(cnt_s)

    p = p_ref[0] + p_ref[1] + y_ref[...]
    h = jnp.maximum(p * dinv_ref[...] * gs_ref[...] + gb_ref[...], 0.0)
    seg = lax.broadcasted_iota(jnp.int32, (BN, G), 1)
    onehot = jnp.where(batch_ref[...] == seg, 1.0, 0.0)
    sums_s[...] += lax.dot_general(onehot, h, (((0,), (0,)), ((), ())),
                                   preferred_element_type=jnp.float32, precision=lax.Precision.HIGHEST)
    cnt_s[...] += lax.dot_general(onehot, jnp.ones((BN, 1), jnp.float32),
                                  (((0,), (0,)), ((), ())),
                                  preferred_element_type=jnp.float32, precision=lax.Precision.HIGHEST)

    @pl.when(i == GRID - 1)
    def _():
        pooled = sums_s[...] / jnp.maximum(cnt_s[...], 1.0)
        r = jnp.maximum(
            jnp.dot(pooled, fc1w_ref[...],
                    preferred_element_type=jnp.float32, precision=lax.Precision.HIGHEST) + fc1b_ref[...], 0.0)
        out_ref[...] = jnp.dot(r, outw_ref[...],
                               preferred_element_type=jnp.float32, precision=lax.Precision.HIGHEST) \
            + outb_ref[...]


def _row_spec(f):
    return pl.BlockSpec((BN, f), lambda i: (i, 0))


def _full_spec(shape):
    return pl.BlockSpec(shape, lambda i: tuple(0 for _ in shape))


def _part_spec(f):
    return pl.BlockSpec((NC, BN, f), lambda i: (0, i, 0))


def _tc1(degp, x, w1):
    return pl.pallas_call(
        _tc1_body,
        grid=(GRID,),
        in_specs=[
            pl.BlockSpec((NC, BN), lambda i: (0, i)),
            _row_spec(D),
            _full_spec((D, 32)),
        ],
        out_specs=[_row_spec(1), _row_spec(32)],
        out_shape=[
            jax.ShapeDtypeStruct((NPAD, 1), jnp.float32),
            jax.ShapeDtypeStruct((NPAD, 32), jnp.float32),
        ],
    )(degp, x, w1)


def _tc_mid(p, y, dinv, gs, gb, wn, f_in, f_out):
    return pl.pallas_call(
        _tc_mid_body,
        grid=(GRID,),
        in_specs=[
            _part_spec(f_in),
            _row_spec(f_in),
            _row_spec(1),
            _full_spec((1, f_in)),
            _full_spec((1, f_in)),
            _full_spec((f_in, f_out)),
        ],
        out_specs=_row_spec(f_out),
        out_shape=jax.ShapeDtypeStruct((NPAD, f_out), jnp.float32),
    )(p, y, dinv, gs, gb, wn)


def _tc_final(p, y, dinv, gs, gb, batch2, fc1w, fc1b, outw, outb):
    return pl.pallas_call(
        _tc_final_body,
        grid=(GRID,),
        in_specs=[
            _part_spec(D),
            _row_spec(D),
            _row_spec(1),
            _full_spec((1, D)),
            _full_spec((1, D)),
            _row_spec(1),
            _full_spec((D, G)),
            _full_spec((1, G)),
            _full_spec((G, 1)),
            _full_spec((1, 1)),
        ],
        out_specs=pl.BlockSpec((G, 1), lambda i: (0, 0)),
        out_shape=jax.ShapeDtypeStruct((G, 1), jnp.float32),
        scratch_shapes=[
            pltpu.VMEM((G, D), jnp.float32),
            pltpu.VMEM((G, 1), jnp.float32),
        ],
    )(p, y, dinv, gs, gb, batch2, fc1w, fc1b, outw, outb)


def kernel(x, edge_index, edge_weight, batch, W1, b1, g1, be1, W2, b2, g2,
           be2, W3, b3, g3, be3, fc1W, fc1b, outW, outb):
    f32 = jnp.float32
    r = edge_index[0].astype(jnp.int32)
    c = edge_index[1].astype(jnp.int32)
    w = edge_weight.astype(f32)

    pad = E_PAD - E
    r3 = jnp.concatenate([r, jnp.zeros((pad,), jnp.int32)]).reshape(
        NW, PER_TILE)
    c3 = jnp.concatenate(
        [c, jnp.full((pad,), PAD_DST, jnp.int32)]).reshape(NW, NCHUNK, CHUNK)
    w3 = jnp.concatenate([w, jnp.zeros((pad,), f32)]).reshape(NW, PER_TILE)

    bnscale = 1.0 / jnp.sqrt(jnp.float32(1.0 + EPS))
    gs1 = (g1 * bnscale).reshape(1, 32)
    gb1 = (b1 * g1 * bnscale + be1).reshape(1, 32)
    gs2 = (g2 * bnscale).reshape(1, 64)
    gb2 = (b2 * g2 * bnscale + be2).reshape(1, 64)
    gs3 = (g3 * bnscale).reshape(1, D)
    gb3 = (b3 * g3 * bnscale + be3).reshape(1, D)
    xp = jnp.concatenate([x, jnp.zeros((NPAD - N, D), f32)])
    batch2 = jnp.concatenate(
        [batch.astype(jnp.int32),
         jnp.full((NPAD - N,), G, jnp.int32)]).reshape(NPAD, 1)

    degp = _make_deg_kernel()(c3, w3)
    dinv, y1 = _tc1(degp, xp, W1)

    p1 = _make_edge_scatter(32)(r3, c3, w3, y1)
    y2 = _tc_mid(p1, y1, dinv, gs1, gb1, W2, 32, 64)

    p2 = _make_edge_scatter(64)(r3, c3, w3, y2)
    y3 = _tc_mid(p2, y2, dinv, gs2, gb2, W3, 64, 128)

    y3a = lax.slice(y3, (0, 0), (NPAD, 64))
    y3b = lax.slice(y3, (0, 64), (NPAD, 128))
    p3a = _make_edge_scatter(64)(r3, c3, w3, y3a)
    p3b = _make_edge_scatter(64)(r3, c3, w3, y3b)
    p3 = jnp.concatenate([p3a, p3b], axis=2)
    return _tc_final(p3, y3, dinv, gs3, gb3, batch2, fc1W,
                     fc1b.reshape(1, G), outW, outb.reshape(1, 1))


# speedup vs baseline: 1.2918x; 1.2918x over previous
import functools
import math

import jax
import jax.numpy as jnp
from jax import lax
from jax.experimental import pallas as pl
from jax.experimental.pallas import tpu as pltpu
from jax.experimental.pallas import tpu_sc as plsc

N = 10000
E = 320000
D = 128
G = 64
EPS = 1e-5

NC = 2
NS = 16
NW = NC * NS

CHUNK = 128
NBUF = 4
NCHUNK = math.ceil(E / NW / CHUNK / NBUF) * NBUF
PER_TILE = NCHUNK * CHUNK
E_PAD = PER_TILE * NW

NPAD = 10240
ROWS_PER_TILE = NPAD // NS
PAD_DST = NPAD - 1

BN = 2048
GRID = NPAD // BN


def _make_mesh():
    return plsc.VectorSubcoreMesh(core_axis_name="c", subcore_axis_name="s",
                                  num_cores=NC, num_subcores=NS)


@functools.cache
def _make_deg_kernel():
    @functools.partial(
        pl.kernel,
        out_type=jax.ShapeDtypeStruct((NC, NPAD), jnp.float32),
        mesh=_make_mesh(),
        scratch_types=[
            pltpu.VMEM((NCHUNK, CHUNK), jnp.int32),
            pltpu.VMEM((PER_TILE,), jnp.float32),
            pltpu.VMEM((ROWS_PER_TILE,), jnp.float32),
            pltpu.VMEM_SHARED((NPAD,), jnp.float32),
        ],
    )
    def _deg_kernel(c_hbm, w_hbm, out_hbm, c_v, w_v, zbuf, acc):
        cid = lax.axis_index("c")
        sid = lax.axis_index("s")
        wid = cid * NS + sid

        pltpu.sync_copy(c_hbm.at[wid], c_v)
        pltpu.sync_copy(w_hbm.at[wid], w_v)

        def _z(i, _):
            zbuf[pl.ds(i * 16, 16)] = jnp.zeros((16,), jnp.float32)
            return 0
        lax.fori_loop(0, ROWS_PER_TILE // 16, _z, 0)
        pltpu.sync_copy(zbuf,
                        acc.at[pl.ds(sid * ROWS_PER_TILE, ROWS_PER_TILE)])
        plsc.subcore_barrier()

        def _body(j, _):
            pltpu.sync_copy(w_v.at[pl.ds(j * CHUNK, CHUNK)],
                            acc.at[c_v.at[j]], add=True)
            return 0
        lax.fori_loop(0, NCHUNK, _body, 0)
        plsc.subcore_barrier()

        sl = pl.ds(sid * ROWS_PER_TILE, ROWS_PER_TILE)
        pltpu.sync_copy(acc.at[sl], out_hbm.at[cid, sl])

    return _deg_kernel


@functools.cache
def _make_edge_scatter(F):
    @functools.partial(
        pl.kernel,
        out_type=jax.ShapeDtypeStruct((NC, NPAD, F), jnp.float32),
        mesh=_make_mesh(),
        compiler_params=pltpu.CompilerParams(needs_layout_passes=False,
                                             use_tc_tiling_on_sc=False),
        scratch_types=[
            pltpu.VMEM((PER_TILE,), jnp.int32),
            pltpu.VMEM((NCHUNK, CHUNK), jnp.int32),
            pltpu.VMEM((PER_TILE,), jnp.float32),
            pltpu.VMEM((NBUF, CHUNK, F), jnp.bfloat16),
            pltpu.VMEM((CHUNK, F), jnp.float32),
            pltpu.VMEM_SHARED((NPAD, F), jnp.float32),
        ] + [pltpu.SemaphoreType.DMA] * NBUF,
    )
    def _edge_kernel(r_hbm, c_hbm, w_hbm, y_hbm, out_hbm,
                     r_v, c_v, w_v, bufs, scat, acc, *sems):
        cid = lax.axis_index("c")
        sid = lax.axis_index("s")
        wid = cid * NS + sid

        pltpu.sync_copy(r_hbm.at[wid], r_v)
        pltpu.sync_copy(c_hbm.at[wid], c_v)
        pltpu.sync_copy(w_hbm.at[wid], w_v)

        def _z(i, _):
            for k in range(F // 16):
                scat[i, pl.ds(k * 16, 16)] = jnp.zeros((16,), jnp.float32)
            return 0
        lax.fori_loop(0, CHUNK, _z, 0)
        for b in range(ROWS_PER_TILE // CHUNK):
            sl = pl.ds(sid * ROWS_PER_TILE + b * CHUNK, CHUNK)
            pltpu.sync_copy(scat, acc.at[sl])
        plsc.subcore_barrier()

        def _gather(j, b):
            r_chunk = r_v.at[pl.ds(j * CHUNK, CHUNK)]
            return pltpu.make_async_copy(y_hbm.at[r_chunk], bufs.at[b],
                                         sems[b])

        for b in range(NBUF):
            _gather(b, b).start()

        def _group(g, _):
            for b in range(NBUF):
                j = g * NBUF + b
                buf = bufs.at[b]
                _gather(j, b).wait()

                def _scale(e, _):
                    idx = jnp.full((16,), j * CHUNK + e, jnp.int32)
                    wv = plsc.load_gather(w_v, [idx])
                    for k in range(F // 32):
                        pk = buf[e, pl.ds(k * 32, 32)]
                        lo, hi = plsc.unpack(
                            pk, format=plsc.PackFormat.INTERLEAVED)
                        scat[e, pl.ds(k * 16, 16)] = lo * wv
                        scat[e, pl.ds(F // 2 + k * 16, 16)] = hi * wv
                    return 0
                lax.fori_loop(0, CHUNK, _scale, 0)

                pltpu.sync_copy(scat, acc.at[c_v.at[j]], add=True)

                @pl.when(j + NBUF < NCHUNK)
                def _():
                    _gather(j + NBUF, b).start()
            return 0
        lax.fori_loop(0, NCHUNK // NBUF, _group, 0)
        plsc.subcore_barrier()

        sl = pl.ds(sid * ROWS_PER_TILE, ROWS_PER_TILE)
        pltpu.sync_copy(acc.at[sl], out_hbm.at[cid, sl])

    return _edge_kernel


def _tc1_body(degp_ref, x_ref, w_ref, ws_ref, dinv_ref, y_ref, ybf_ref):
    ones = jnp.ones((NC, 1), jnp.float32)
    deg = lax.dot_general(degp_ref[...], ones, (((0,), (0,)), ((), ())),
                          preferred_element_type=jnp.float32, precision=lax.Precision.HIGHEST) + 1.0
    dinv = lax.rsqrt(deg)
    xw = jnp.dot(x_ref[...], w_ref[...], preferred_element_type=jnp.float32, precision=lax.Precision.HIGHEST)
    xws = jnp.dot(x_ref[...], ws_ref[...], preferred_element_type=jnp.float32, precision=lax.Precision.HIGHEST)
    dinv_ref[...] = dinv
    y_ref[...] = xw * dinv
    ybf_ref[...] = (xws * dinv).astype(jnp.bfloat16)


def _tc_mid_body(p_ref, y_ref, dinv_ref, gs_ref, gb_ref, w_ref, ws_ref,
                 out_ref, outbf_ref):
    p = p_ref[0] + p_ref[1] + y_ref[...]
    dinv = dinv_ref[...]
    h = jnp.maximum(p * dinv * gs_ref[...] + gb_ref[...], 0.0)
    out_ref[...] = jnp.dot(h, w_ref[...],
                           preferred_element_type=jnp.float32, precision=lax.Precision.HIGHEST) * dinv
    outbf_ref[...] = (jnp.dot(h, ws_ref[...],
                              preferred_element_type=jnp.float32, precision=lax.Precision.HIGHEST)
                      * dinv).astype(jnp.bfloat16)


def _tc_final_body(p_ref, y_ref, dinv_ref, gs_ref, gb_ref, batch_ref,
                   fc1w_ref, fc1b_ref, outw_ref, outb_ref, out_ref,
                   sums_s, cnt_s):
    i = pl.program_id(0)

    @pl.when(i == 0)
    def _():
        sums_s[...] = jnp.zeros_like(sums_s)
        cnt_s[...] = jnp.zeros_like(cnt_s)

    p = p_ref[0] + p_ref[1] + y_ref[...]
    h = jnp.maximum(p * dinv_ref[...] * gs_ref[...] + gb_ref[...], 0.0)
    seg = lax.broadcasted_iota(jnp.int32, (BN, G), 1)
    onehot = jnp.where(batch_ref[...] == seg, 1.0, 0.0)
    sums_s[...] += lax.dot_general(onehot, h, (((0,), (0,)), ((), ())),
                                   preferred_element_type=jnp.float32, precision=lax.Precision.HIGHEST)
    cnt_s[...] += lax.dot_general(onehot, jnp.ones((BN, 1), jnp.float32),
                                  (((0,), (0,)), ((), ())),
                                  preferred_element_type=jnp.float32, precision=lax.Precision.HIGHEST)

    @pl.when(i == GRID - 1)
    def _():
        pooled = sums_s[...] / jnp.maximum(cnt_s[...], 1.0)
        r = jnp.maximum(
            jnp.dot(pooled, fc1w_ref[...],
                    preferred_element_type=jnp.float32, precision=lax.Precision.HIGHEST) + fc1b_ref[...], 0.0)
        out_ref[...] = jnp.dot(r, outw_ref[...],
                               preferred_element_type=jnp.float32, precision=lax.Precision.HIGHEST) \
            + outb_ref[...]


def _row_spec(f):
    return pl.BlockSpec((BN, f), lambda i: (i, 0))


def _full_spec(shape):
    return pl.BlockSpec(shape, lambda i: tuple(0 for _ in shape))


def _part_spec(f):
    return pl.BlockSpec((NC, BN, f), lambda i: (0, i, 0))


def _tc1(degp, x, w1, w1s):
    return pl.pallas_call(
        _tc1_body,
        grid=(GRID,),
        in_specs=[
            pl.BlockSpec((NC, BN), lambda i: (0, i)),
            _row_spec(D),
            _full_spec((D, 32)),
            _full_spec((D, 32)),
        ],
        out_specs=[_row_spec(1), _row_spec(32), _row_spec(32)],
        out_shape=[
            jax.ShapeDtypeStruct((NPAD, 1), jnp.float32),
            jax.ShapeDtypeStruct((NPAD, 32), jnp.float32),
            jax.ShapeDtypeStruct((NPAD, 32), jnp.bfloat16),
        ],
    )(degp, x, w1, w1s)


def _tc_mid(p, y, dinv, gs, gb, wn, wns, f_in, f_out):
    return pl.pallas_call(
        _tc_mid_body,
        grid=(GRID,),
        in_specs=[
            _part_spec(f_in),
            _row_spec(f_in),
            _row_spec(1),
            _full_spec((1, f_in)),
            _full_spec((1, f_in)),
            _full_spec((f_in, f_out)),
            _full_spec((f_in, f_out)),
        ],
        out_specs=[_row_spec(f_out), _row_spec(f_out)],
        out_shape=[
            jax.ShapeDtypeStruct((NPAD, f_out), jnp.float32),
            jax.ShapeDtypeStruct((NPAD, f_out), jnp.bfloat16),
        ],
    )(p, y, dinv, gs, gb, wn, wns)


def _tc_final(p, y, dinv, gs, gb, batch2, fc1w, fc1b, outw, outb):
    return pl.pallas_call(
        _tc_final_body,
        grid=(GRID,),
        in_specs=[
            _part_spec(D),
            _row_spec(D),
            _row_spec(1),
            _full_spec((1, D)),
            _full_spec((1, D)),
            _row_spec(1),
            _full_spec((D, G)),
            _full_spec((1, G)),
            _full_spec((G, 1)),
            _full_spec((1, 1)),
        ],
        out_specs=pl.BlockSpec((G, 1), lambda i: (0, 0)),
        out_shape=jax.ShapeDtypeStruct((G, 1), jnp.float32),
        scratch_shapes=[
            pltpu.VMEM((G, D), jnp.float32),
            pltpu.VMEM((G, 1), jnp.float32),
        ],
    )(p, y, dinv, gs, gb, batch2, fc1w, fc1b, outw, outb)


def kernel(x, edge_index, edge_weight, batch, W1, b1, g1, be1, W2, b2, g2,
           be2, W3, b3, g3, be3, fc1W, fc1b, outW, outb):
    f32 = jnp.float32
    r = edge_index[0].astype(jnp.int32)
    c = edge_index[1].astype(jnp.int32)
    w = edge_weight.astype(f32)

    pad = E_PAD - E
    r3 = jnp.concatenate([r, jnp.zeros((pad,), jnp.int32)]).reshape(
        NW, PER_TILE)
    c3 = jnp.concatenate(
        [c, jnp.full((pad,), PAD_DST, jnp.int32)]).reshape(NW, NCHUNK, CHUNK)
    w3 = jnp.concatenate([w, jnp.zeros((pad,), f32)]).reshape(NW, PER_TILE)

    bnscale = 1.0 / jnp.sqrt(jnp.float32(1.0 + EPS))
    gs1 = (g1 * bnscale).reshape(1, 32)
    gb1 = (b1 * g1 * bnscale + be1).reshape(1, 32)
    gs2 = (g2 * bnscale).reshape(1, 64)
    gb2 = (b2 * g2 * bnscale + be2).reshape(1, 64)
    gs3 = (g3 * bnscale).reshape(1, D)
    gb3 = (b3 * g3 * bnscale + be3).reshape(1, D)
    xp = jnp.concatenate([x, jnp.zeros((NPAD - N, D), f32)])
    batch2 = jnp.concatenate(
        [batch.astype(jnp.int32),
         jnp.full((NPAD - N,), G, jnp.int32)]).reshape(NPAD, 1)

    def _perm(f):
        import numpy as _np
        p = _np.empty((f,), _np.int32)
        p[0::2] = _np.arange(f // 2)
        p[1::2] = _np.arange(f // 2) + f // 2
        return p
    pm32, pm64 = _perm(32), _perm(64)
    W1s = W1[:, pm32]
    W2s = W2[:, pm64]
    W3s = jnp.concatenate([W3[:, :64][:, pm64], W3[:, 64:][:, pm64]], axis=1)

    degp = _make_deg_kernel()(c3, w3)
    dinv, y1, y1bf = _tc1(degp, xp, W1, W1s)

    p1 = _make_edge_scatter(32)(r3, c3, w3, y1bf)
    y2, y2bf = _tc_mid(p1, y1, dinv, gs1, gb1, W2, W2s, 32, 64)

    p2 = _make_edge_scatter(64)(r3, c3, w3, y2bf)
    y3, y3bf = _tc_mid(p2, y2, dinv, gs2, gb2, W3, W3s, 64, 128)

    y3abf = lax.slice(y3bf, (0, 0), (NPAD, 64))
    y3bbf = lax.slice(y3bf, (0, 64), (NPAD, 128))
    p3a = _make_edge_scatter(64)(r3, c3, w3, y3abf)
    p3b = _make_edge_scatter(64)(r3, c3, w3, y3bbf)
    p3 = jnp.concatenate([p3a, p3b], axis=2)
    return _tc_final(p3, y3, dinv, gs3, gb3, batch2, fc1W,
                     fc1b.reshape(1, G), outW, outb.reshape(1, 1))


# speedup vs baseline: 1.5839x; 1.2261x over previous
import functools
import math

import jax
import jax.numpy as jnp
from jax import lax
from jax.experimental import pallas as pl
from jax.experimental.pallas import tpu as pltpu
from jax.experimental.pallas import tpu_sc as plsc

N = 10000
E = 320000
D = 128
G = 64
EPS = 1e-5

NC = 2
NS = 16
NW = NC * NS

CHUNK = 128
NBUF = 2
NCHUNK = math.ceil(E / NW / CHUNK / NBUF) * NBUF
PER_TILE = NCHUNK * CHUNK
E_PAD = PER_TILE * NW

NPAD = 10240
ROWS_PER_TILE = NPAD // NS
PAD_DST = NPAD - 1

BN = 2048
GRID = NPAD // BN


def _make_mesh():
    return plsc.VectorSubcoreMesh(core_axis_name="c", subcore_axis_name="s",
                                  num_cores=NC, num_subcores=NS)


@functools.cache
def _make_deg_kernel():
    @functools.partial(
        pl.kernel,
        out_type=jax.ShapeDtypeStruct((NC, NPAD), jnp.float32),
        mesh=_make_mesh(),
        scratch_types=[
            pltpu.VMEM((NCHUNK, CHUNK), jnp.int32),
            pltpu.VMEM((PER_TILE,), jnp.float32),
            pltpu.VMEM((ROWS_PER_TILE,), jnp.float32),
            pltpu.VMEM_SHARED((NPAD,), jnp.float32),
        ],
    )
    def _deg_kernel(c_hbm, w_hbm, out_hbm, c_v, w_v, zbuf, acc):
        cid = lax.axis_index("c")
        sid = lax.axis_index("s")
        wid = cid * NS + sid

        pltpu.sync_copy(c_hbm.at[wid], c_v)
        pltpu.sync_copy(w_hbm.at[wid], w_v)

        def _z(i, _):
            zbuf[pl.ds(i * 16, 16)] = jnp.zeros((16,), jnp.float32)
            return 0
        lax.fori_loop(0, ROWS_PER_TILE // 16, _z, 0)
        pltpu.sync_copy(zbuf,
                        acc.at[pl.ds(sid * ROWS_PER_TILE, ROWS_PER_TILE)])
        plsc.subcore_barrier()

        def _body(j, _):
            pltpu.sync_copy(w_v.at[pl.ds(j * CHUNK, CHUNK)],
                            acc.at[c_v.at[j]], add=True)
            return 0
        lax.fori_loop(0, NCHUNK, _body, 0)
        plsc.subcore_barrier()

        sl = pl.ds(sid * ROWS_PER_TILE, ROWS_PER_TILE)
        pltpu.sync_copy(acc.at[sl], out_hbm.at[cid, sl])

    return _deg_kernel


@functools.cache
def _make_edge_scatter(F):
    @functools.partial(
        pl.kernel,
        out_type=jax.ShapeDtypeStruct((NC, NPAD, F), jnp.float32),
        mesh=_make_mesh(),
        compiler_params=pltpu.CompilerParams(needs_layout_passes=False,
                                             use_tc_tiling_on_sc=False),
        scratch_types=[
            pltpu.VMEM((PER_TILE,), jnp.int32),
            pltpu.VMEM((NCHUNK, CHUNK), jnp.int32),
            pltpu.VMEM((PER_TILE,), jnp.float32),
            pltpu.VMEM((NBUF, CHUNK, F), jnp.float32),
            pltpu.VMEM_SHARED((NPAD, F), jnp.float32),
            pltpu.VMEM_SHARED((NPAD, F), jnp.float32),
        ] + [pltpu.SemaphoreType.DMA] * NBUF,
    )
    def _edge_kernel(r_hbm, c_hbm, w_hbm, y_hbm, out_hbm,
                     r_v, c_v, w_v, bufs, ytab, acc, *sems):
        cid = lax.axis_index("c")
        sid = lax.axis_index("s")
        wid = cid * NS + sid

        pltpu.sync_copy(r_hbm.at[wid], r_v)
        pltpu.sync_copy(c_hbm.at[wid], c_v)
        pltpu.sync_copy(w_hbm.at[wid], w_v)

        tsl = pl.ds(sid * ROWS_PER_TILE, ROWS_PER_TILE)
        pltpu.sync_copy(y_hbm.at[tsl], ytab.at[tsl])

        buf0 = bufs.at[0]

        def _z(i, _):
            for k in range(F // 16):
                buf0[i, pl.ds(k * 16, 16)] = jnp.zeros((16,), jnp.float32)
            return 0
        lax.fori_loop(0, CHUNK, _z, 0)
        for b in range(ROWS_PER_TILE // CHUNK):
            sl = pl.ds(sid * ROWS_PER_TILE + b * CHUNK, CHUNK)
            pltpu.sync_copy(buf0, acc.at[sl])
        plsc.subcore_barrier()

        def _gather(j, b):
            r_chunk = r_v.at[pl.ds(j * CHUNK, CHUNK)]
            return pltpu.make_async_copy(ytab.at[r_chunk], bufs.at[b],
                                         sems[b])

        for b in range(NBUF):
            _gather(b, b).start()

        def _group(g, _):
            for b in range(NBUF):
                j = g * NBUF + b
                buf = bufs.at[b]
                _gather(j, b).wait()

                def _scale(e, _):
                    idx = jnp.full((16,), j * CHUNK + e, jnp.int32)
                    wv = plsc.load_gather(w_v, [idx])
                    for k in range(F // 16):
                        sl = pl.ds(k * 16, 16)
                        buf[e, sl] = buf[e, sl] * wv
                    return 0
                lax.fori_loop(0, CHUNK, _scale, 0)

                pltpu.sync_copy(buf, acc.at[c_v.at[j]], add=True)

                @pl.when(j + NBUF < NCHUNK)
                def _():
                    _gather(j + NBUF, b).start()
            return 0
        lax.fori_loop(0, NCHUNK // NBUF, _group, 0)
        plsc.subcore_barrier()

        sl = pl.ds(sid * ROWS_PER_TILE, ROWS_PER_TILE)
        pltpu.sync_copy(acc.at[sl], out_hbm.at[cid, sl])

    return _edge_kernel


def _tc1_body(degp_ref, x_ref, w_ref, dinv_ref, y_ref):
    ones = jnp.ones((NC, 1), jnp.float32)
    deg = lax.dot_general(degp_ref[...], ones, (((0,), (0,)), ((), ())),
                          preferred_element_type=jnp.float32, precision=lax.Precision.HIGHEST) + 1.0
    dinv = lax.rsqrt(deg)
    xw = jnp.dot(x_ref[...], w_ref[...], preferred_element_type=jnp.float32, precision=lax.Precision.HIGHEST)
    dinv_ref[...] = dinv
    y_ref[...] = xw * dinv


def _tc_mid_body(p_ref, y_ref, dinv_ref, gs_ref, gb_ref, w_ref, out_ref):
    p = p_ref[0] + p_ref[1] + y_ref[...]
    dinv = dinv_ref[...]
    h = jnp.maximum(p * dinv * gs_ref[...] + gb_ref[...], 0.0)
    out_ref[...] = jnp.dot(h, w_ref[...],
                           preferred_element_type=jnp.float32, precision=lax.Precision.HIGHEST) * dinv


def _tc_final_body(p_ref, y_ref, dinv_ref, gs_ref, gb_ref, batch_ref,
                   fc1w_ref, fc1b_ref, outw_ref, outb_ref, out_ref,
                   sums_s, cnt_s):
    i = pl.program_id(0)

    @pl.when(i == 0)
    def _():
        sums_s[...] = jnp.zeros_like(sums_s)
        cnt_s[...] = jnp.zeros_like(cnt_s)

    p = p_ref[0] + p_ref[1] + y_ref[...]
    h = jnp.maximum(p * dinv_ref[...] * gs_ref[...] + gb_ref[...], 0.0)
    seg = lax.broadcasted_iota(jnp.int32, (BN, G), 1)
    onehot = jnp.where(batch_ref[...] == seg, 1.0, 0.0)
    sums_s[...] += lax.dot_general(onehot, h, (((0,), (0,)), ((), ())),
                                   preferred_element_type=jnp.float32, precision=lax.Precision.HIGHEST)
    cnt_s[...] += lax.dot_general(onehot, jnp.ones((BN, 1), jnp.float32),
                                  (((0,), (0,)), ((), ())),
                                  preferred_element_type=jnp.float32, precision=lax.Precision.HIGHEST)

    @pl.when(i == GRID - 1)
    def _():
        pooled = sums_s[...] / jnp.maximum(cnt_s[...], 1.0)
        r = jnp.maximum(
            jnp.dot(pooled, fc1w_ref[...],
                    preferred_element_type=jnp.float32, precision=lax.Precision.HIGHEST) + fc1b_ref[...], 0.0)
        out_ref[...] = jnp.dot(r, outw_ref[...],
                               preferred_element_type=jnp.float32, precision=lax.Precision.HIGHEST) \
            + outb_ref[...]


def _row_spec(f):
    return pl.BlockSpec((BN, f), lambda i: (i, 0))


def _full_spec(shape):
    return pl.BlockSpec(shape, lambda i: tuple(0 for _ in shape))


def _part_spec(f):
    return pl.BlockSpec((NC, BN, f), lambda i: (0, i, 0))


def _tc1(degp, x, w1):
    return pl.pallas_call(
        _tc1_body,
        grid=(GRID,),
        in_specs=[
            pl.BlockSpec((NC, BN), lambda i: (0, i)),
            _row_spec(D),
            _full_spec((D, 32)),
        ],
        out_specs=[_row_spec(1), _row_spec(32)],
        out_shape=[
            jax.ShapeDtypeStruct((NPAD, 1), jnp.float32),
            jax.ShapeDtypeStruct((NPAD, 32), jnp.float32),
        ],
    )(degp, x, w1)


def _tc_mid(p, y, dinv, gs, gb, wn, f_in, f_out):
    return pl.pallas_call(
        _tc_mid_body,
        grid=(GRID,),
        in_specs=[
            _part_spec(f_in),
            _row_spec(f_in),
            _row_spec(1),
            _full_spec((1, f_in)),
            _full_spec((1, f_in)),
            _full_spec((f_in, f_out)),
        ],
        out_specs=_row_spec(f_out),
        out_shape=jax.ShapeDtypeStruct((NPAD, f_out), jnp.float32),
    )(p, y, dinv, gs, gb, wn)


def _tc_final(p, y, dinv, gs, gb, batch2, fc1w, fc1b, outw, outb):
    return pl.pallas_call(
        _tc_final_body,
        grid=(GRID,),
        in_specs=[
            _part_spec(D),
            _row_spec(D),
            _row_spec(1),
            _full_spec((1, D)),
            _full_spec((1, D)),
            _row_spec(1),
            _full_spec((D, G)),
            _full_spec((1, G)),
            _full_spec((G, 1)),
            _full_spec((1, 1)),
        ],
        out_specs=pl.BlockSpec((G, 1), lambda i: (0, 0)),
        out_shape=jax.ShapeDtypeStruct((G, 1), jnp.float32),
        scratch_shapes=[
            pltpu.VMEM((G, D), jnp.float32),
            pltpu.VMEM((G, 1), jnp.float32),
        ],
    )(p, y, dinv, gs, gb, batch2, fc1w, fc1b, outw, outb)


def kernel(x, edge_index, edge_weight, batch, W1, b1, g1, be1, W2, b2, g2,
           be2, W3, b3, g3, be3, fc1W, fc1b, outW, outb):
    f32 = jnp.float32
    r = edge_index[0].astype(jnp.int32)
    c = edge_index[1].astype(jnp.int32)
    w = edge_weight.astype(f32)

    pad = E_PAD - E
    r3 = jnp.concatenate([r, jnp.zeros((pad,), jnp.int32)]).reshape(
        NW, PER_TILE)
    c3 = jnp.concatenate(
        [c, jnp.full((pad,), PAD_DST, jnp.int32)]).reshape(NW, NCHUNK, CHUNK)
    w3 = jnp.concatenate([w, jnp.zeros((pad,), f32)]).reshape(NW, PER_TILE)

    bnscale = 1.0 / jnp.sqrt(jnp.float32(1.0 + EPS))
    gs1 = (g1 * bnscale).reshape(1, 32)
    gb1 = (b1 * g1 * bnscale + be1).reshape(1, 32)
    gs2 = (g2 * bnscale).reshape(1, 64)
    gb2 = (b2 * g2 * bnscale + be2).reshape(1, 64)
    gs3 = (g3 * bnscale).reshape(1, D)
    gb3 = (b3 * g3 * bnscale + be3).reshape(1, D)
    xp = jnp.concatenate([x, jnp.zeros((NPAD - N, D), f32)])
    batch2 = jnp.concatenate(
        [batch.astype(jnp.int32),
         jnp.full((NPAD - N,), G, jnp.int32)]).reshape(NPAD, 1)

    degp = _make_deg_kernel()(c3, w3)
    dinv, y1 = _tc1(degp, xp, W1)

    p1 = _make_edge_scatter(32)(r3, c3, w3, y1)
    y2 = _tc_mid(p1, y1, dinv, gs1, gb1, W2, 32, 64)

    p2 = _make_edge_scatter(64)(r3, c3, w3, y2)
    y3 = _tc_mid(p2, y2, dinv, gs2, gb2, W3, 64, 128)

    y3a = lax.slice(y3, (0, 0), (NPAD, 64))
    y3b = lax.slice(y3, (0, 64), (NPAD, 128))
    p3a = _make_edge_scatter(64)(r3, c3, w3, y3a)
    p3b = _make_edge_scatter(64)(r3, c3, w3, y3b)
    p3 = jnp.concatenate([p3a, p3b], axis=2)
    return _tc_final(p3, y3, dinv, gs3, gb3, batch2, fc1W,
                     fc1b.reshape(1, G), outW, outb.reshape(1, 1))
